# baseline (speedup 1.0000x reference)
import jax
import jax.numpy as jnp
import numpy as np
from jax.experimental import pallas as pl
from jax.experimental.pallas import tpu as pltpu

H = 224
W = 224
N = H * W
C = 96
HEADS = 8
D = 12
G = C // 16
TN = 7168
P = 512
RB = TN // P
NB = N // P
TPB = N // TN
TNE = TN + 2 * P

WA = TN + 2 * 256
AO = P - 256
WS = TN + 2 * 128
VO = P - 128

OFFSETS = [(dr, dc) for dr in (-1, 0, 1) for dc in (-1, 0, 1)]
NEG = -1e30

_PERM = np.array([(cp % HEADS) * D + cp // HEADS for cp in range(C)])
_IPERM = np.array([(c % D) * HEADS + c // D for c in range(C)])


def _gat_grid_kernel(xl_ref, xc_ref, xr_ref, wt_ref, asrc_ref, adst_ref,
                     bias_ref, lnw_ref, lnb_ref, out_ref):
    i = pl.program_id(1)
    j0 = i * TN

    x_ext = jnp.concatenate([xl_ref[0], xc_ref[0], xr_ref[0]], axis=1)
    xw_ext = jnp.dot(wt_ref[:], x_ext.astype(jnp.bfloat16),
                     preferred_element_type=jnp.float32)
    s_ext = jnp.dot(asrc_ref[:], xw_ext, preferred_element_type=jnp.float32)
    t = jnp.dot(adst_ref[:], xw_ext[:, AO:AO + WA],
                preferred_element_type=jnp.float32)

    idxa = jax.lax.broadcasted_iota(jnp.int32, (1, WA), 1) + (j0 - 256 + 448)
    r = idxa // W - 2
    c = idxa % W

    a_list = []
    m = jnp.full((HEADS, WA), NEG, jnp.float32)
    for dr, dc in OFFSETS:
        delta = dr * W + dc
        sk = s_ext[:, AO - delta:AO - delta + WA]
        z = sk + t
        a = jnp.where(z >= 0, z, 0.2 * z)
        if dr != 0 or dc != 0:
            mask = None
            if dr != 0:
                rs = r - dr
                mask = (rs >= 0) & (rs < H)
            if dc != 0:
                cs = c - dc
                mc = (cs >= 0) & (cs < W)
                mask = mc if mask is None else (mask & mc)
            a = jnp.where(mask, a, NEG)
        a_list.append(a)
        m = jnp.maximum(m, a)

    den = jnp.zeros((HEADS, WA), jnp.float32)
    e_all = {}
    for (dr, dc), a in zip(OFFSETS, a_list):
        e = jnp.exp(a - m)
        if dr == 0 and dc == 0:
            e = e * 2.0
        e_all[(dr, dc)] = e
        den = den + e
    inv = 1.0 / (den + 1e-16)

    xwb = xw_ext.astype(jnp.bfloat16).reshape(G, 16, TNE)
    xrow = {dr: xwb[:, :, VO - dr * W:VO - dr * W + WS] for dr in (-1, 0, 1)}

    acc = None
    for dc in (-1, 0, 1):
        v = None
        for dr in (-1, 0, 1):
            wp = ((e_all[(dr, dc)] * inv)[:, 128 + dc:128 + dc + WS]
                  ).astype(jnp.bfloat16)
            w16 = jnp.concatenate([wp, wp], axis=0)[None, :, :]
            term = w16 * xrow[dr]
            v = term if v is None else v + term
        vc = v[:, :, 128 - dc:128 - dc + TN]
        acc = vc if acc is None else acc + vc

    o = acc.reshape(C, TN).astype(jnp.float32) + bias_ref[:]
    o = jnp.where(o > 0, o, jnp.exp(o) - 1.0)
    mu = jnp.mean(o, axis=0, keepdims=True)
    m2 = jnp.mean(o * o, axis=0, keepdims=True)
    isd = jax.lax.rsqrt(m2 - mu * mu + 1e-5)
    nmu = -mu * isd
    o = o * isd + nmu
    o = o * lnw_ref[:] + lnb_ref[:]
    out_ref[0] = o


def kernel(x, Wlin, att_src, att_dst, bias, ln_w, ln_b):
    B = x.shape[0]
    x3 = x.reshape(B, C, N)
    perm = jnp.asarray(_PERM)
    wt = Wlin.T[perm, :].astype(jnp.bfloat16)
    eye = jnp.eye(HEADS, dtype=jnp.float32)
    a_src = (att_src[:, None, :] * eye[:, :, None]).reshape(HEADS, C)
    a_dst = (att_dst[:, None, :] * eye[:, :, None]).reshape(HEADS, C)
    a_src = a_src[:, perm]
    a_dst = a_dst[:, perm]

    out = pl.pallas_call(
        _gat_grid_kernel,
        grid=(B, TPB),
        in_specs=[
            pl.BlockSpec((1, C, P),
                         lambda b, i: (b, 0, jnp.maximum(i * RB - 1, 0))),
            pl.BlockSpec((1, C, TN), lambda b, i: (b, 0, i)),
            pl.BlockSpec((1, C, P),
                         lambda b, i: (b, 0, jnp.minimum(i * RB + RB, NB - 1))),
            pl.BlockSpec((C, C), lambda b, i: (0, 0)),
            pl.BlockSpec((HEADS, C), lambda b, i: (0, 0)),
            pl.BlockSpec((HEADS, C), lambda b, i: (0, 0)),
            pl.BlockSpec((C, 1), lambda b, i: (0, 0)),
            pl.BlockSpec((C, 1), lambda b, i: (0, 0)),
            pl.BlockSpec((C, 1), lambda b, i: (0, 0)),
        ],
        out_specs=pl.BlockSpec((1, C, TN), lambda b, i: (b, 0, i)),
        out_shape=jax.ShapeDtypeStruct((B, C, N), jnp.float32),
        compiler_params=pltpu.CompilerParams(
            dimension_semantics=("parallel", "parallel")),
    )(x3, x3, x3, wt, a_src, a_dst,
      bias[perm].reshape(C, 1), ln_w[perm].reshape(C, 1),
      ln_b[perm].reshape(C, 1))
    return out[:, jnp.asarray(_IPERM), :].reshape(B, C, H, W)


# speedup vs baseline: 1.3178x; 1.3178x over previous
import jax
import jax.numpy as jnp
import numpy as np
from jax.experimental import pallas as pl
from jax.experimental.pallas import tpu as pltpu

H = 224
W = 224
N = H * W
C = 96
HEADS = 8
D = 12
G = C // 16
TN = 7168
P = 512
RB = TN // P
NB = N // P
TPB = N // TN
TNE = TN + 2 * P

WA = TN + 2 * 256
AO = P - 256
WS = TN + 2 * 128
VO = P - 128

OFFSETS = [(dr, dc) for dr in (-1, 0, 1) for dc in (-1, 0, 1)]
NEG = -1e30

_PERM = np.array([(cp % HEADS) * D + cp // HEADS for cp in range(C)])
_IPERM = np.array([(c % D) * HEADS + c // D for c in range(C)])


def _gat_grid_kernel(xl_ref, xc_ref, xr_ref, wt_ref, asrc_ref, adst_ref,
                     bias_ref, lnw_ref, lnb_ref, out_ref):
    i = pl.program_id(1)
    j0 = i * TN

    x_ext = jnp.concatenate([xl_ref[0], xc_ref[0], xr_ref[0]], axis=1)
    xw_ext = jnp.dot(wt_ref[:], x_ext.astype(jnp.bfloat16),
                     preferred_element_type=jnp.float32)
    s_ext = jnp.dot(asrc_ref[:], xw_ext, preferred_element_type=jnp.float32)
    t = jnp.dot(adst_ref[:], xw_ext[:, AO:AO + WA],
                preferred_element_type=jnp.float32)

    idxa = jax.lax.broadcasted_iota(jnp.int32, (1, WA), 1) + (j0 - 256 + 448)
    r = idxa // W - 2
    c = idxa % W

    a_list = []
    m = jnp.full((HEADS, WA), NEG, jnp.float32)
    for dr, dc in OFFSETS:
        delta = dr * W + dc
        sk = s_ext[:, AO - delta:AO - delta + WA]
        z = sk + t
        a = jnp.where(z >= 0, z, 0.2 * z)
        if dr != 0 or dc != 0:
            mask = None
            if dr != 0:
                rs = r - dr
                mask = (rs >= 0) & (rs < H)
            if dc != 0:
                cs = c - dc
                mc = (cs >= 0) & (cs < W)
                mask = mc if mask is None else (mask & mc)
            a = jnp.where(mask, a, NEG)
        a_list.append(a)
        m = jnp.maximum(m, a)

    den = jnp.zeros((HEADS, WA), jnp.float32)
    e_all = {}
    for (dr, dc), a in zip(OFFSETS, a_list):
        e = jnp.exp(a - m)
        if dr == 0 and dc == 0:
            e = e * 2.0
        e_all[(dr, dc)] = e
        den = den + e
    inv = 1.0 / (den + 1e-16)

    xwb = xw_ext.astype(jnp.bfloat16).reshape(G, 16, TNE)
    xrow = {dr: xwb[:, :, VO - dr * W:VO - dr * W + WS] for dr in (-1, 0, 1)}

    acc = None
    for dc in (-1, 0, 1):
        v = None
        for dr in (-1, 0, 1):
            wp = ((e_all[(dr, dc)] * inv)[:, 128 + dc:128 + dc + WS]
                  ).astype(jnp.bfloat16)
            w16 = jnp.concatenate([wp, wp], axis=0)[None, :, :]
            term = w16 * xrow[dr]
            v = term if v is None else v + term
        vc = v[:, :, 128 - dc:128 - dc + TN]
        acc = vc if acc is None else acc + vc

    o = acc.reshape(C, TN).astype(jnp.float32) + bias_ref[:]
    o = jnp.where(o > 0, o, jnp.exp(o) - 1.0)
    mu = jnp.mean(o, axis=0, keepdims=True)
    m2 = jnp.mean(o * o, axis=0, keepdims=True)
    isd = jax.lax.rsqrt(m2 - mu * mu + 1e-5)
    nmu = -mu * isd
    o = o * isd + nmu
    o = o * lnw_ref[:] + lnb_ref[:]
    out_ref[0] = o


def kernel(x, Wlin, att_src, att_dst, bias, ln_w, ln_b):
    B = x.shape[0]
    x3 = x.reshape(B, C, N)
    perm = jnp.asarray(_PERM)
    wt = Wlin.T[perm, :].astype(jnp.bfloat16)
    eye = jnp.eye(HEADS, dtype=jnp.float32)
    a_src = (att_src[:, None, :] * eye[:, :, None]).reshape(HEADS, C)
    a_dst = (att_dst[:, None, :] * eye[:, :, None]).reshape(HEADS, C)
    a_src = a_src[:, perm]
    a_dst = a_dst[:, perm]

    out = pl.pallas_call(
        _gat_grid_kernel,
        grid=(B, TPB),
        in_specs=[
            pl.BlockSpec((1, C, P),
                         lambda b, i: (b, 0, jnp.maximum(i * RB - 1, 0))),
            pl.BlockSpec((1, C, TN), lambda b, i: (b, 0, i)),
            pl.BlockSpec((1, C, P),
                         lambda b, i: (b, 0, jnp.minimum(i * RB + RB, NB - 1))),
            pl.BlockSpec((C, C), lambda b, i: (0, 0)),
            pl.BlockSpec((HEADS, C), lambda b, i: (0, 0)),
            pl.BlockSpec((HEADS, C), lambda b, i: (0, 0)),
            pl.BlockSpec((C, 1), lambda b, i: (0, 0)),
            pl.BlockSpec((C, 1), lambda b, i: (0, 0)),
            pl.BlockSpec((C, 1), lambda b, i: (0, 0)),
        ],
        out_specs=pl.BlockSpec((1, C, TN), lambda b, i: (b, 0, i)),
        out_shape=jax.ShapeDtypeStruct((B, C, N), jnp.float32),
        compiler_params=pltpu.CompilerParams(
            dimension_semantics=("parallel", "parallel")),
    )(x3, x3, x3, wt, a_src, a_dst,
      bias[perm].reshape(C, 1), ln_w[perm].reshape(C, 1),
      ln_b[perm].reshape(C, 1))
    return (out.reshape(B, D, HEADS, N).transpose(0, 2, 1, 3)
            .reshape(B, C, H, W))


# speedup vs baseline: 1.9346x; 1.4680x over previous
import jax
import jax.numpy as jnp
from jax.experimental import pallas as pl
from jax.experimental.pallas import tpu as pltpu

H = 224
W = 224
N = H * W
C = 96
HEADS = 8
D = 12
TN = 12544
P = 1792
RB = TN // P
NB = N // P
TPB = N // TN
TNE = TN + 2 * P

WA = TN + 2 * 256
AO = P - 256
WS = TN + 2 * 128
VO = P - 128

OFFSETS = [(dr, dc) for dr in (-1, 0, 1) for dc in (-1, 0, 1)]
NEG = -1e30


def _gat_grid_kernel(xl_ref, xc_ref, xr_ref, wt_ref, asrc_ref, adst_ref,
                     bias_ref, lnw_ref, lnb_ref, out_ref):
    i = pl.program_id(1)
    j0 = i * TN

    x_ext = jnp.concatenate([xl_ref[0], xc_ref[0], xr_ref[0]], axis=1)
    xw_ext = jnp.dot(wt_ref[:], x_ext.astype(jnp.bfloat16),
                     preferred_element_type=jnp.float32)
    s_ext = jnp.dot(asrc_ref[:], xw_ext, preferred_element_type=jnp.float32)
    t = jnp.dot(adst_ref[:], xw_ext[:, AO:AO + WA],
                preferred_element_type=jnp.float32)

    idxa = jax.lax.broadcasted_iota(jnp.int32, (1, WA), 1) + (j0 - 256 + 448)
    r = idxa // W - 2
    c = idxa % W

    a_list = []
    m = jnp.full((HEADS, WA), NEG, jnp.float32)
    for dr, dc in OFFSETS:
        delta = dr * W + dc
        sk = s_ext[:, AO - delta:AO - delta + WA]
        z = sk + t
        a = jnp.where(z >= 0, z, 0.2 * z)
        if dr != 0 or dc != 0:
            mask = None
            if dr != 0:
                rs = r - dr
                mask = (rs >= 0) & (rs < H)
            if dc != 0:
                cs = c - dc
                mc = (cs >= 0) & (cs < W)
                mask = mc if mask is None else (mask & mc)
            a = jnp.where(mask, a, NEG)
        a_list.append(a)
        m = jnp.maximum(m, a)

    den = jnp.zeros((HEADS, WA), jnp.float32)
    e_all = {}
    for (dr, dc), a in zip(OFFSETS, a_list):
        e = jnp.exp(a - m)
        if dr == 0 and dc == 0:
            e = e * 2.0
        e_all[(dr, dc)] = e
        den = den + e
    inv = 1.0 / (den + 1e-16)

    rep = (jax.lax.broadcasted_iota(jnp.int32, (C, HEADS), 0) // D ==
           jax.lax.broadcasted_iota(jnp.int32, (C, HEADS), 1)
           ).astype(jnp.bfloat16)

    xwb = xw_ext.astype(jnp.bfloat16)
    xrow = {dr: xwb[:, VO - dr * W:VO - dr * W + WS] for dr in (-1, 0, 1)}

    acc = None
    for dc in (-1, 0, 1):
        wcat = jnp.concatenate(
            [(e_all[(dr, dc)] * inv)[:, 128 + dc:128 + dc + WS]
             for dr in (-1, 0, 1)], axis=1)
        wfull = jnp.dot(rep, wcat.astype(jnp.bfloat16),
                        preferred_element_type=jnp.float32
                        ).astype(jnp.bfloat16)
        v = (wfull[:, 0:WS] * xrow[-1]
             + wfull[:, WS:2 * WS] * xrow[0]
             + wfull[:, 2 * WS:3 * WS] * xrow[1])
        vc = v[:, 128 - dc:128 - dc + TN]
        acc = vc if acc is None else acc + vc

    o = acc.astype(jnp.float32) + bias_ref[:]
    o = jnp.where(o > 0, o, jnp.exp(o) - 1.0)
    mu = jnp.mean(o, axis=0, keepdims=True)
    m2 = jnp.mean(o * o, axis=0, keepdims=True)
    isd = jax.lax.rsqrt(m2 - mu * mu + 1e-5)
    nmu = -mu * isd
    o = o * isd + nmu
    o = o * lnw_ref[:] + lnb_ref[:]
    out_ref[0] = o


def kernel(x, Wlin, att_src, att_dst, bias, ln_w, ln_b):
    B = x.shape[0]
    x3 = x.reshape(B, C, N)
    wt = Wlin.T.astype(jnp.bfloat16)
    eye = jnp.eye(HEADS, dtype=jnp.float32)
    a_src = (att_src[:, None, :] * eye[:, :, None]).reshape(HEADS, C)
    a_dst = (att_dst[:, None, :] * eye[:, :, None]).reshape(HEADS, C)

    out = pl.pallas_call(
        _gat_grid_kernel,
        grid=(B, TPB),
        in_specs=[
            pl.BlockSpec((1, C, P),
                         lambda b, i: (b, 0, jnp.maximum(i * RB - 1, 0))),
            pl.BlockSpec((1, C, TN), lambda b, i: (b, 0, i)),
            pl.BlockSpec((1, C, P),
                         lambda b, i: (b, 0, jnp.minimum(i * RB + RB, NB - 1))),
            pl.BlockSpec((C, C), lambda b, i: (0, 0)),
            pl.BlockSpec((HEADS, C), lambda b, i: (0, 0)),
            pl.BlockSpec((HEADS, C), lambda b, i: (0, 0)),
            pl.BlockSpec((C, 1), lambda b, i: (0, 0)),
            pl.BlockSpec((C, 1), lambda b, i: (0, 0)),
            pl.BlockSpec((C, 1), lambda b, i: (0, 0)),
        ],
        out_specs=pl.BlockSpec((1, C, TN), lambda b, i: (b, 0, i)),
        out_shape=jax.ShapeDtypeStruct((B, C, N), jnp.float32),
        compiler_params=pltpu.CompilerParams(
            dimension_semantics=("parallel", "parallel")),
    )(x3, x3, x3, wt, a_src, a_dst,
      bias.reshape(C, 1), ln_w.reshape(C, 1), ln_b.reshape(C, 1))
    return out.reshape(B, C, H, W)
